# baseline (speedup 1.0000x reference)
import functools

import jax
import jax.numpy as jnp
from jax import lax
from jax.experimental import pallas as pl
from jax.experimental.pallas import tpu as pltpu
from jax.experimental.pallas import tpu_sc as plsc

N = 10000
E = 160000
NC = 2
NS = 16
NW = NC * NS
K = 128
CPS = 80
TOT_CH = NS * CPS
E_PAD = TOT_CH * K
ACC_ROWS = 10112
RPS = ACC_ROWS // NS
BR = 1000


def _make_propagate(D, f_ch):
  s_ch = CPS - f_ch
  mesh = plsc.VectorSubcoreMesh(core_axis_name="c", subcore_axis_name="s")

  @functools.partial(
      pl.kernel,
      mesh=mesh,
      compiler_params=pltpu.CompilerParams(use_tc_tiling_on_sc=False),
      out_type=jax.ShapeDtypeStruct((NC, ACC_ROWS, D), jnp.float32),
      scratch_types=[
          pltpu.VMEM((f_ch, K), jnp.int32),
          pltpu.VMEM((f_ch, K), jnp.int32),
          pltpu.VMEM((2, K, D), jnp.float32),
          pltpu.VMEM_SHARED((ACC_ROWS, D), jnp.float32),
          pltpu.SemaphoreType.DMA,
          pltpu.SemaphoreType.DMA,
      ],
  )
  def prop(table_hbm, src_hbm, dst_hbm, zeros_hbm, out_hbm,
           src_v, dst_v, rows_v, acc_sh, gsem0, gsem1):
    c = lax.axis_index("c")
    s = lax.axis_index("s")
    st = jnp.where(c == 0, s * f_ch, NS * f_ch + s * s_ch)
    nch = jnp.where(c == 0, f_ch, s_ch)

    @pl.when(c == 0)
    def _():
      pltpu.sync_copy(src_hbm.at[pl.ds(st, f_ch)], src_v)
      pltpu.sync_copy(dst_hbm.at[pl.ds(st, f_ch)], dst_v)

    @pl.when(c != 0)
    def _():
      pltpu.sync_copy(src_hbm.at[pl.ds(st, s_ch)], src_v.at[pl.ds(0, s_ch)])
      pltpu.sync_copy(dst_hbm.at[pl.ds(st, s_ch)], dst_v.at[pl.ds(0, s_ch)])

    pltpu.async_copy(table_hbm.at[src_v.at[0]], rows_v.at[0], gsem0)
    pltpu.async_copy(table_hbm.at[src_v.at[1]], rows_v.at[1], gsem1)
    pltpu.sync_copy(zeros_hbm, acc_sh.at[pl.ds(s * RPS, RPS)])
    plsc.subcore_barrier()

    def body(i, carry):
      jj = i * 2
      for b, sem in ((0, gsem0), (1, gsem1)):
        j = jj + b
        pltpu.make_async_copy(
            table_hbm.at[src_v.at[j]], rows_v.at[b], sem).wait()
        pltpu.sync_copy(rows_v.at[b], acc_sh.at[dst_v.at[j]], add=True)

        @pl.when(j + 2 < nch)
        def _():
          pltpu.async_copy(table_hbm.at[src_v.at[j + 2]], rows_v.at[b], sem)

      return carry

    lax.fori_loop(0, nch // 2, body, 0)
    plsc.subcore_barrier()
    pltpu.sync_copy(acc_sh.at[pl.ds(s * RPS, RPS)],
                    out_hbm.at[c, pl.ds(s * RPS, RPS)])

  return prop


_prop16 = _make_propagate(16, 64)
_prop128 = _make_propagate(128, 64)


def _rsqrt_deg(da_ref, db_ref):
  deg = da_ref[:, 0:1] + db_ref[:, 0:1] + 1.0
  return lax.rsqrt(deg)


def _tc1(x_ref, da_ref, db_ref, w_ref, o_ref):
  s = _rsqrt_deg(da_ref, db_ref)
  h = jnp.dot(x_ref[...], w_ref[...], preferred_element_type=jnp.float32)
  o_ref[...] = h * s


def _tc2(aa_ref, ab_ref, g1_ref, da_ref, db_ref, w_ref, b_ref, o_ref):
  s = _rsqrt_deg(da_ref, db_ref)
  h = s * (aa_ref[...] + ab_ref[...] + g1_ref[...]) + b_ref[...]
  h = jnp.maximum(h, 0.0)
  h2 = jnp.dot(h, w_ref[...], preferred_element_type=jnp.float32)
  o_ref[...] = h2 * s


def _tc3(aa_ref, ab_ref, g2_ref, da_ref, db_ref, b_ref, o_ref):
  s = _rsqrt_deg(da_ref, db_ref)
  t = s * (aa_ref[...] + ab_ref[...] + g2_ref[...]) + b_ref[...]
  col = lax.broadcasted_iota(jnp.int32, t.shape, 1)
  valid = col < 12
  t = jnp.where(valid, t, -jnp.inf)
  m = jnp.max(t, axis=1, keepdims=True)
  e = jnp.where(valid, jnp.exp(t - m), 0.0)
  lse = jnp.log(jnp.sum(e, axis=1, keepdims=True))
  o_ref[...] = (t - m - lse)[:, :12]


def _row_spec(d):
  return pl.BlockSpec((BR, d), lambda i: (i, 0))


def _full_spec(r, c):
  return pl.BlockSpec((r, c), lambda i: (0, 0))


def kernel(x, edge_index, W1, b1, W2, b2):
  x = x.astype(jnp.float32)
  ei = edge_index.astype(jnp.int32)
  pad = E_PAD - E
  src = jnp.concatenate([ei[0], jnp.zeros((pad,), jnp.int32)])
  dummy_dst = N + jnp.arange(pad, dtype=jnp.int32) % (ACC_ROWS - N)
  dst = jnp.concatenate([ei[1], dummy_dst])
  src3 = src.reshape(TOT_CH, K)
  dst3 = dst.reshape(TOT_CH, K)
  z16 = jnp.zeros((RPS, 16), jnp.float32)
  z128 = jnp.zeros((RPS, 128), jnp.float32)
  ones_tab = jnp.ones((N, 16), jnp.float32)

  degp = _prop16(ones_tab, src3, dst3, z16)
  dega = degp[0, :N]
  degb = degp[1, :N]

  grid = N // BR
  g1 = pl.pallas_call(
      _tc1,
      grid=(grid,),
      in_specs=[_row_spec(256), _row_spec(16), _row_spec(16),
                _full_spec(256, 128)],
      out_specs=_row_spec(128),
      out_shape=jax.ShapeDtypeStruct((N, 128), jnp.float32),
  )(x, dega, degb, W1)

  acc1 = _prop128(g1, src3, dst3, z128)

  W2p = jnp.pad(W2.astype(jnp.float32), ((0, 0), (0, 4)))
  b1r = b1.astype(jnp.float32).reshape(1, 128)
  g2 = pl.pallas_call(
      _tc2,
      grid=(grid,),
      in_specs=[_row_spec(128), _row_spec(128), _row_spec(128),
                _row_spec(16), _row_spec(16), _full_spec(128, 16),
                _full_spec(1, 128)],
      out_specs=_row_spec(16),
      out_shape=jax.ShapeDtypeStruct((N, 16), jnp.float32),
  )(acc1[0, :N], acc1[1, :N], g1, dega, degb, W2p, b1r)

  acc2 = _prop16(g2, src3, dst3, z16)

  b2r = jnp.pad(b2.astype(jnp.float32), (0, 4)).reshape(1, 16)
  out = pl.pallas_call(
      _tc3,
      grid=(grid,),
      in_specs=[_row_spec(16), _row_spec(16), _row_spec(16),
                _row_spec(16), _row_spec(16), _full_spec(1, 16)],
      out_specs=pl.BlockSpec((BR, 12), lambda i: (i, 0)),
      out_shape=jax.ShapeDtypeStruct((N, 12), jnp.float32),
  )(acc2[0, :N], acc2[1, :N], g2, dega, degb, b2r)

  return out


# speedup vs baseline: 1.1750x; 1.1750x over previous
import functools

import jax
import jax.numpy as jnp
from jax import lax
from jax.experimental import pallas as pl
from jax.experimental.pallas import tpu as pltpu
from jax.experimental.pallas import tpu_sc as plsc

N = 10000
E = 160000
NC = 2
NS = 16
NW = NC * NS
K = 128
CPS = 80
TOT_CH = NS * CPS
E_PAD = TOT_CH * K
ACC_ROWS = 10112
RPS = ACC_ROWS // NS
BR = 1000


def _make_propagate(D, f_ch):
  s_ch = CPS - f_ch
  mesh = plsc.VectorSubcoreMesh(core_axis_name="c", subcore_axis_name="s")

  @functools.partial(
      pl.kernel,
      mesh=mesh,
      compiler_params=pltpu.CompilerParams(use_tc_tiling_on_sc=False),
      out_type=jax.ShapeDtypeStruct((NC, ACC_ROWS, D), jnp.float32),
      scratch_types=[
          pltpu.VMEM((f_ch, K), jnp.int32),
          pltpu.VMEM((f_ch, K), jnp.int32),
          pltpu.VMEM((2, K, D), jnp.float32),
          pltpu.VMEM_SHARED((ACC_ROWS, D), jnp.float32),
          pltpu.SemaphoreType.DMA,
          pltpu.SemaphoreType.DMA,
      ],
  )
  def prop(table_hbm, src_hbm, dst_hbm, zeros_hbm, out_hbm,
           src_v, dst_v, rows_v, acc_sh, gsem0, gsem1):
    c = lax.axis_index("c")
    s = lax.axis_index("s")
    st = jnp.where(c == 0, s * f_ch, NS * f_ch + s * s_ch)
    nch = jnp.where(c == 0, f_ch, s_ch)

    @pl.when(c == 0)
    def _():
      pltpu.sync_copy(src_hbm.at[pl.ds(st, f_ch)], src_v)
      pltpu.sync_copy(dst_hbm.at[pl.ds(st, f_ch)], dst_v)

    @pl.when(c != 0)
    def _():
      pltpu.sync_copy(src_hbm.at[pl.ds(st, s_ch)], src_v.at[pl.ds(0, s_ch)])
      pltpu.sync_copy(dst_hbm.at[pl.ds(st, s_ch)], dst_v.at[pl.ds(0, s_ch)])

    pltpu.async_copy(table_hbm.at[src_v.at[0]], rows_v.at[0], gsem0)
    pltpu.async_copy(table_hbm.at[src_v.at[1]], rows_v.at[1], gsem1)
    pltpu.sync_copy(zeros_hbm, acc_sh.at[pl.ds(s * RPS, RPS)])
    plsc.subcore_barrier()

    def body(i, carry):
      jj = i * 2
      for b, sem in ((0, gsem0), (1, gsem1)):
        j = jj + b
        pltpu.make_async_copy(
            table_hbm.at[src_v.at[j]], rows_v.at[b], sem).wait()
        pltpu.sync_copy(rows_v.at[b], acc_sh.at[dst_v.at[j]], add=True)

        @pl.when(j + 2 < nch)
        def _():
          pltpu.async_copy(table_hbm.at[src_v.at[j + 2]], rows_v.at[b], sem)

      return carry

    lax.fori_loop(0, nch // 2, body, 0)
    plsc.subcore_barrier()
    pltpu.sync_copy(acc_sh.at[pl.ds(s * RPS, RPS)],
                    out_hbm.at[c, pl.ds(s * RPS, RPS)])

  return prop


_prop16 = _make_propagate(16, 64)


def _make_propagate_colsplit():
  D = 64
  mesh = plsc.VectorSubcoreMesh(core_axis_name="c", subcore_axis_name="s")

  @functools.partial(
      pl.kernel,
      mesh=mesh,
      compiler_params=pltpu.CompilerParams(use_tc_tiling_on_sc=False),
      out_type=jax.ShapeDtypeStruct((NC, ACC_ROWS, D), jnp.float32),
      scratch_types=[
          pltpu.VMEM((CPS, K), jnp.int32),
          pltpu.VMEM((CPS, K), jnp.int32),
          pltpu.VMEM((2, K, D), jnp.float32),
          pltpu.VMEM_SHARED((ACC_ROWS, D), jnp.float32),
          pltpu.SemaphoreType.DMA,
          pltpu.SemaphoreType.DMA,
      ],
  )
  def prop(t0_hbm, t1_hbm, src_hbm, dst_hbm, zeros_hbm, out_hbm,
           src_v, dst_v, rows_v, acc_sh, gsem0, gsem1):
    c = lax.axis_index("c")
    s = lax.axis_index("s")
    st = s * CPS

    pltpu.sync_copy(src_hbm.at[pl.ds(st, CPS)], src_v)
    pltpu.sync_copy(dst_hbm.at[pl.ds(st, CPS)], dst_v)

    def run(table_hbm):
      pltpu.async_copy(table_hbm.at[src_v.at[0]], rows_v.at[0], gsem0)
      pltpu.async_copy(table_hbm.at[src_v.at[1]], rows_v.at[1], gsem1)
      pltpu.sync_copy(zeros_hbm, acc_sh.at[pl.ds(s * RPS, RPS)])
      plsc.subcore_barrier()

      def body(i, carry):
        jj = i * 2
        for b, sem in ((0, gsem0), (1, gsem1)):
          j = jj + b
          pltpu.make_async_copy(
              table_hbm.at[src_v.at[j]], rows_v.at[b], sem).wait()
          pltpu.sync_copy(rows_v.at[b], acc_sh.at[dst_v.at[j]], add=True)

          @pl.when(j + 2 < CPS)
          def _():
            pltpu.async_copy(table_hbm.at[src_v.at[j + 2]], rows_v.at[b], sem)

        return carry

      lax.fori_loop(0, CPS // 2, body, 0)
      plsc.subcore_barrier()

    @pl.when(c == 0)
    def _():
      run(t0_hbm)

    @pl.when(c != 0)
    def _():
      run(t1_hbm)

    pltpu.sync_copy(acc_sh.at[pl.ds(s * RPS, RPS)],
                    out_hbm.at[c, pl.ds(s * RPS, RPS)])

  return prop


_prop128 = _make_propagate_colsplit()


def _rsqrt_deg(da_ref, db_ref):
  deg = da_ref[:, 0:1] + db_ref[:, 0:1] + 1.0
  return lax.rsqrt(deg)


def _tc1(x_ref, da_ref, db_ref, w_ref, o0_ref, o1_ref):
  s = _rsqrt_deg(da_ref, db_ref)
  h = jnp.dot(x_ref[...], w_ref[...], preferred_element_type=jnp.float32)
  g = h * s
  o0_ref[...] = g[:, :64]
  o1_ref[...] = g[:, 64:]


def _tc2(aa_ref, ab_ref, g0_ref, g1_ref, da_ref, db_ref, w_ref, b_ref,
         o_ref):
  s = _rsqrt_deg(da_ref, db_ref)
  ha = s * (aa_ref[...] + g0_ref[...]) + b_ref[:, :64]
  hb = s * (ab_ref[...] + g1_ref[...]) + b_ref[:, 64:]
  h = jnp.maximum(jnp.concatenate([ha, hb], axis=1), 0.0)
  h2 = jnp.dot(h, w_ref[...], preferred_element_type=jnp.float32)
  o_ref[...] = h2 * s


def _tc3(aa_ref, ab_ref, g2_ref, da_ref, db_ref, b_ref, o_ref):
  s = _rsqrt_deg(da_ref, db_ref)
  t = s * (aa_ref[...] + ab_ref[...] + g2_ref[...]) + b_ref[...]
  col = lax.broadcasted_iota(jnp.int32, t.shape, 1)
  valid = col < 12
  t = jnp.where(valid, t, -jnp.inf)
  m = jnp.max(t, axis=1, keepdims=True)
  e = jnp.where(valid, jnp.exp(t - m), 0.0)
  lse = jnp.log(jnp.sum(e, axis=1, keepdims=True))
  o_ref[...] = (t - m - lse)[:, :12]


def _row_spec(d):
  return pl.BlockSpec((BR, d), lambda i: (i, 0))


def _full_spec(r, c):
  return pl.BlockSpec((r, c), lambda i: (0, 0))


def kernel(x, edge_index, W1, b1, W2, b2):
  x = x.astype(jnp.float32)
  ei = edge_index.astype(jnp.int32)
  pad = E_PAD - E
  src = jnp.concatenate([ei[0], jnp.zeros((pad,), jnp.int32)])
  dummy_dst = N + jnp.arange(pad, dtype=jnp.int32) % (ACC_ROWS - N)
  dst = jnp.concatenate([ei[1], dummy_dst])
  src3 = src.reshape(TOT_CH, K)
  dst3 = dst.reshape(TOT_CH, K)
  z16 = jnp.zeros((RPS, 16), jnp.float32)
  z64 = jnp.zeros((RPS, 64), jnp.float32)
  ones_tab = jnp.ones((N, 16), jnp.float32)

  degp = _prop16(ones_tab, src3, dst3, z16)
  dega = degp[0, :N]
  degb = degp[1, :N]

  grid = N // BR
  g1a, g1b = pl.pallas_call(
      _tc1,
      grid=(grid,),
      in_specs=[_row_spec(256), _row_spec(16), _row_spec(16),
                _full_spec(256, 128)],
      out_specs=(_row_spec(64), _row_spec(64)),
      out_shape=(jax.ShapeDtypeStruct((N, 64), jnp.float32),
                 jax.ShapeDtypeStruct((N, 64), jnp.float32)),
  )(x, dega, degb, W1)

  acc1 = _prop128(g1a, g1b, src3, dst3, z64)

  W2p = jnp.pad(W2.astype(jnp.float32), ((0, 0), (0, 4)))
  b1r = b1.astype(jnp.float32).reshape(1, 128)
  g2 = pl.pallas_call(
      _tc2,
      grid=(grid,),
      in_specs=[_row_spec(64), _row_spec(64), _row_spec(64), _row_spec(64),
                _row_spec(16), _row_spec(16), _full_spec(128, 16),
                _full_spec(1, 128)],
      out_specs=_row_spec(16),
      out_shape=jax.ShapeDtypeStruct((N, 16), jnp.float32),
  )(acc1[0, :N], acc1[1, :N], g1a, g1b, dega, degb, W2p, b1r)

  acc2 = _prop16(g2, src3, dst3, z16)

  b2r = jnp.pad(b2.astype(jnp.float32), (0, 4)).reshape(1, 16)
  out = pl.pallas_call(
      _tc3,
      grid=(grid,),
      in_specs=[_row_spec(16), _row_spec(16), _row_spec(16),
                _row_spec(16), _row_spec(16), _full_spec(1, 16)],
      out_specs=pl.BlockSpec((BR, 12), lambda i: (i, 0)),
      out_shape=jax.ShapeDtypeStruct((N, 12), jnp.float32),
  )(acc2[0, :N], acc2[1, :N], g2, dega, degb, b2r)

  return out


# speedup vs baseline: 1.1946x; 1.0167x over previous
import functools

import jax
import jax.numpy as jnp
from jax import lax
from jax.experimental import pallas as pl
from jax.experimental.pallas import tpu as pltpu
from jax.experimental.pallas import tpu_sc as plsc

N = 10000
E = 160000
NC = 2
NS = 16
NW = NC * NS
K = 128
CPS = 80
TOT_CH = NS * CPS
E_PAD = TOT_CH * K
ACC_ROWS = 10112
RPS = ACC_ROWS // NS
BR = 1000


def _make_propagate(D, f_ch):
  s_ch = CPS - f_ch
  mesh = plsc.VectorSubcoreMesh(core_axis_name="c", subcore_axis_name="s")

  @functools.partial(
      pl.kernel,
      mesh=mesh,
      compiler_params=pltpu.CompilerParams(use_tc_tiling_on_sc=False),
      out_type=jax.ShapeDtypeStruct((NC, ACC_ROWS, D), jnp.float32),
      scratch_types=[
          pltpu.VMEM((f_ch, K), jnp.int32),
          pltpu.VMEM((f_ch, K), jnp.int32),
          pltpu.VMEM((2, K, D), jnp.float32),
          pltpu.VMEM_SHARED((ACC_ROWS, D), jnp.float32),
          pltpu.SemaphoreType.DMA,
          pltpu.SemaphoreType.DMA,
      ],
  )
  def prop(table_hbm, src_hbm, dst_hbm, zeros_hbm, out_hbm,
           src_v, dst_v, rows_v, acc_sh, gsem0, gsem1):
    c = lax.axis_index("c")
    s = lax.axis_index("s")
    st = jnp.where(c == 0, s * f_ch, NS * f_ch + s * s_ch)
    nch = jnp.where(c == 0, f_ch, s_ch)

    @pl.when(c == 0)
    def _():
      pltpu.sync_copy(src_hbm.at[pl.ds(st, f_ch)], src_v)
      pltpu.sync_copy(dst_hbm.at[pl.ds(st, f_ch)], dst_v)

    @pl.when(c != 0)
    def _():
      pltpu.sync_copy(src_hbm.at[pl.ds(st, s_ch)], src_v.at[pl.ds(0, s_ch)])
      pltpu.sync_copy(dst_hbm.at[pl.ds(st, s_ch)], dst_v.at[pl.ds(0, s_ch)])

    pltpu.async_copy(table_hbm.at[src_v.at[0]], rows_v.at[0], gsem0)
    pltpu.async_copy(table_hbm.at[src_v.at[1]], rows_v.at[1], gsem1)
    pltpu.sync_copy(zeros_hbm, acc_sh.at[pl.ds(s * RPS, RPS)])
    plsc.subcore_barrier()

    def body(i, carry):
      jj = i * 2
      for b, sem in ((0, gsem0), (1, gsem1)):
        j = jj + b
        pltpu.make_async_copy(
            table_hbm.at[src_v.at[j]], rows_v.at[b], sem).wait()
        pltpu.sync_copy(rows_v.at[b], acc_sh.at[dst_v.at[j]], add=True)

        @pl.when(j + 2 < nch)
        def _():
          pltpu.async_copy(table_hbm.at[src_v.at[j + 2]], rows_v.at[b], sem)

      return carry

    lax.fori_loop(0, nch // 2, body, 0)
    plsc.subcore_barrier()
    pltpu.sync_copy(acc_sh.at[pl.ds(s * RPS, RPS)],
                    out_hbm.at[c, pl.ds(s * RPS, RPS)])

  return prop


_prop16 = _make_propagate(16, 64)


def _make_propagate_colsplit():
  D = 64
  mesh = plsc.VectorSubcoreMesh(core_axis_name="c", subcore_axis_name="s")

  @functools.partial(
      pl.kernel,
      mesh=mesh,
      compiler_params=pltpu.CompilerParams(use_tc_tiling_on_sc=False),
      out_type=jax.ShapeDtypeStruct((NC, ACC_ROWS, D), jnp.float32),
      scratch_types=[
          pltpu.VMEM((CPS, K), jnp.int32),
          pltpu.VMEM((CPS, K), jnp.int32),
          pltpu.VMEM((4, K, D), jnp.float32),
          pltpu.VMEM_SHARED((ACC_ROWS, D), jnp.float32),
          pltpu.SemaphoreType.DMA,
          pltpu.SemaphoreType.DMA,
          pltpu.SemaphoreType.DMA,
          pltpu.SemaphoreType.DMA,
      ],
  )
  def prop(t0_hbm, t1_hbm, src_hbm, dst_hbm, zeros_hbm, out_hbm,
           src_v, dst_v, rows_v, acc_sh, gsem0, gsem1, gsem2, gsem3):
    c = lax.axis_index("c")
    s = lax.axis_index("s")
    st = s * CPS

    pltpu.sync_copy(src_hbm.at[pl.ds(st, CPS)], src_v)
    pltpu.sync_copy(dst_hbm.at[pl.ds(st, CPS)], dst_v)

    def run(table_hbm):
      sems = (gsem0, gsem1, gsem2, gsem3)
      for b in range(4):
        pltpu.async_copy(table_hbm.at[src_v.at[b]], rows_v.at[b], sems[b])
      pltpu.sync_copy(zeros_hbm, acc_sh.at[pl.ds(s * RPS, RPS)])
      plsc.subcore_barrier()

      def body(i, carry):
        jj = i * 4
        for b in range(4):
          j = jj + b
          sem = sems[b]
          pltpu.make_async_copy(
              table_hbm.at[src_v.at[j]], rows_v.at[b], sem).wait()
          pltpu.sync_copy(rows_v.at[b], acc_sh.at[dst_v.at[j]], add=True)

          @pl.when(j + 4 < CPS)
          def _():
            pltpu.async_copy(table_hbm.at[src_v.at[j + 4]], rows_v.at[b], sem)

        return carry

      lax.fori_loop(0, CPS // 4, body, 0)
      plsc.subcore_barrier()

    @pl.when(c == 0)
    def _():
      run(t0_hbm)

    @pl.when(c != 0)
    def _():
      run(t1_hbm)

    pltpu.sync_copy(acc_sh.at[pl.ds(s * RPS, RPS)],
                    out_hbm.at[c, pl.ds(s * RPS, RPS)])

  return prop


_prop128 = _make_propagate_colsplit()


def _rsqrt_deg(da_ref, db_ref):
  deg = da_ref[:, 0:1] + db_ref[:, 0:1] + 1.0
  return lax.rsqrt(deg)


def _tc1(x_ref, da_ref, db_ref, w_ref, o0_ref, o1_ref):
  s = _rsqrt_deg(da_ref, db_ref)
  h = jnp.dot(x_ref[...], w_ref[...], preferred_element_type=jnp.float32)
  g = h * s
  o0_ref[...] = g[:, :64]
  o1_ref[...] = g[:, 64:]


def _tc2(aa_ref, ab_ref, g0_ref, g1_ref, da_ref, db_ref, w_ref, b_ref,
         o_ref):
  s = _rsqrt_deg(da_ref, db_ref)
  ha = s * (aa_ref[...] + g0_ref[...]) + b_ref[:, :64]
  hb = s * (ab_ref[...] + g1_ref[...]) + b_ref[:, 64:]
  h = jnp.maximum(jnp.concatenate([ha, hb], axis=1), 0.0)
  h2 = jnp.dot(h, w_ref[...], preferred_element_type=jnp.float32)
  o_ref[...] = h2 * s


def _tc3(aa_ref, ab_ref, g2_ref, da_ref, db_ref, b_ref, o_ref):
  s = _rsqrt_deg(da_ref, db_ref)
  t = s * (aa_ref[...] + ab_ref[...] + g2_ref[...]) + b_ref[...]
  col = lax.broadcasted_iota(jnp.int32, t.shape, 1)
  valid = col < 12
  t = jnp.where(valid, t, -jnp.inf)
  m = jnp.max(t, axis=1, keepdims=True)
  e = jnp.where(valid, jnp.exp(t - m), 0.0)
  lse = jnp.log(jnp.sum(e, axis=1, keepdims=True))
  o_ref[...] = (t - m - lse)[:, :12]


def _row_spec(d):
  return pl.BlockSpec((BR, d), lambda i: (i, 0))


def _full_spec(r, c):
  return pl.BlockSpec((r, c), lambda i: (0, 0))


def kernel(x, edge_index, W1, b1, W2, b2):
  x = x.astype(jnp.float32)
  ei = edge_index.astype(jnp.int32)
  pad = E_PAD - E
  src = jnp.concatenate([ei[0], jnp.zeros((pad,), jnp.int32)])
  dummy_dst = N + jnp.arange(pad, dtype=jnp.int32) % (ACC_ROWS - N)
  dst = jnp.concatenate([ei[1], dummy_dst])
  src3 = src.reshape(TOT_CH, K)
  dst3 = dst.reshape(TOT_CH, K)
  z16 = jnp.zeros((RPS, 16), jnp.float32)
  z64 = jnp.zeros((RPS, 64), jnp.float32)
  ones_tab = jnp.ones((N, 16), jnp.float32)

  degp = _prop16(ones_tab, src3, dst3, z16)
  dega = degp[0, :N]
  degb = degp[1, :N]

  grid = N // BR
  g1a, g1b = pl.pallas_call(
      _tc1,
      grid=(grid,),
      in_specs=[_row_spec(256), _row_spec(16), _row_spec(16),
                _full_spec(256, 128)],
      out_specs=(_row_spec(64), _row_spec(64)),
      out_shape=(jax.ShapeDtypeStruct((N, 64), jnp.float32),
                 jax.ShapeDtypeStruct((N, 64), jnp.float32)),
  )(x, dega, degb, W1)

  acc1 = _prop128(g1a, g1b, src3, dst3, z64)

  W2p = jnp.pad(W2.astype(jnp.float32), ((0, 0), (0, 4)))
  b1r = b1.astype(jnp.float32).reshape(1, 128)
  g2 = pl.pallas_call(
      _tc2,
      grid=(grid,),
      in_specs=[_row_spec(64), _row_spec(64), _row_spec(64), _row_spec(64),
                _row_spec(16), _row_spec(16), _full_spec(128, 16),
                _full_spec(1, 128)],
      out_specs=_row_spec(16),
      out_shape=jax.ShapeDtypeStruct((N, 16), jnp.float32),
  )(acc1[0, :N], acc1[1, :N], g1a, g1b, dega, degb, W2p, b1r)

  acc2 = _prop16(g2, src3, dst3, z16)

  b2r = jnp.pad(b2.astype(jnp.float32), (0, 4)).reshape(1, 16)
  out = pl.pallas_call(
      _tc3,
      grid=(grid,),
      in_specs=[_row_spec(16), _row_spec(16), _row_spec(16),
                _row_spec(16), _row_spec(16), _full_spec(1, 16)],
      out_specs=pl.BlockSpec((BR, 12), lambda i: (i, 0)),
      out_shape=jax.ShapeDtypeStruct((N, 12), jnp.float32),
  )(acc2[0, :N], acc2[1, :N], g2, dega, degb, b2r)

  return out
